# baseline (speedup 1.0000x reference)
import functools

import jax
import jax.numpy as jnp
from jax import lax
from jax.experimental import pallas as pl
from jax.experimental.pallas import tpu as pltpu
from jax.experimental.pallas import tpu_sc as plsc

_EMB = 64
_SCALE = 8.0
_NC, _NS, _L = 2, 16, 16
_NW = _NC * _NS
_T = 200
_BATCH = 4096
_BB = _BATCH // _NW

_mesh = plsc.VectorSubcoreMesh(core_axis_name="c", subcore_axis_name="s")


@functools.partial(
    pl.kernel,
    out_type=jax.ShapeDtypeStruct((_T, _EMB, _BATCH), jnp.float32),
    mesh=_mesh,
    scratch_types=[
        pltpu.VMEM((_T, _BB), jnp.int32),
        pltpu.VMEM((2, _BB, _EMB), jnp.float32),
        pltpu.VMEM((2, _EMB, _BB), jnp.float32),
        pltpu.SemaphoreType.DMA,
        pltpu.SemaphoreType.DMA,
        pltpu.SemaphoreType.DMA,
        pltpu.SemaphoreType.DMA,
    ],
    compiler_params=pltpu.CompilerParams(
        use_tc_tiling_on_sc=False, needs_layout_passes=False
    ),
)
def _emb_lookup(table_hbm, tok_hbm, out_hbm, idx_v, rows_v, outt_v,
                gsem0, gsem1, osem0, osem1):
    gsem = (gsem0, gsem1)
    osem = (osem0, osem1)
    wid = lax.axis_index("s") * _NC + lax.axis_index("c")
    b0 = wid * _BB

    pltpu.sync_copy(tok_hbm.at[:, pl.ds(b0, _BB)], idx_v)

    iota = lax.iota(jnp.int32, _L)
    rowsel = [iota + (j * _L) for j in range(_BB // _L)]

    def start_gather(t, b):
        pltpu.make_async_copy(
            table_hbm.at[idx_v.at[t]], rows_v.at[b], gsem[b]
        ).start()

    def wait_gather(t, b):
        pltpu.make_async_copy(
            table_hbm.at[idx_v.at[t]], rows_v.at[b], gsem[b]
        ).wait()

    def start_out(t, b):
        pltpu.make_async_copy(
            outt_v.at[b], out_hbm.at[t, :, pl.ds(b0, _BB)], osem[b]
        ).start()

    def wait_out(t, b):
        pltpu.make_async_copy(
            outt_v.at[b], out_hbm.at[t, :, pl.ds(b0, _BB)], osem[b]
        ).wait()

    start_gather(0, 0)
    start_gather(1, 1)

    @pl.loop(0, _T, step=2)
    def _pair(t0):
        for b in range(2):
            t = t0 + b
            wait_gather(t, b)

            @pl.when(t >= 2)
            def _():
                wait_out(t - 2, b)

            rows = rows_v.at[b]
            outt = outt_v.at[b]

            @pl.loop(0, _EMB)
            def _col(c):
                colv = jnp.full((_L,), 0, jnp.int32) + c
                for j in range(_BB // _L):
                    v = plsc.load_gather(rows, [rowsel[j], colv])
                    outt[c, pl.ds(j * _L, _L)] = v * _SCALE

            start_out(t, b)

            @pl.when(t + 2 < _T)
            def _():
                start_gather(t + 2, b)

    wait_out(_T - 2, 0)
    wait_out(_T - 1, 1)


def kernel(tokens, table):
    out_t = _emb_lookup(table, tokens.T)
    return jnp.transpose(out_t, (2, 0, 1))


# speedup vs baseline: 1.4680x; 1.4680x over previous
import functools

import jax
import jax.numpy as jnp
from jax import lax
from jax.experimental import pallas as pl
from jax.experimental.pallas import tpu as pltpu
from jax.experimental.pallas import tpu_sc as plsc

_EMB = 64
_SCALE = 8.0
_NC, _NS, _L = 2, 16, 16
_NW = _NC * _NS
_T = 200
_BATCH = 4096
_BB = _BATCH // _NW

_mesh = plsc.VectorSubcoreMesh(core_axis_name="c", subcore_axis_name="s")


@functools.partial(
    pl.kernel,
    out_type=jax.ShapeDtypeStruct((_T, _EMB, _BATCH), jnp.float32),
    mesh=_mesh,
    scratch_types=[
        pltpu.VMEM((_T, _BB), jnp.int32),
        pltpu.VMEM((2, _BB, _EMB), jnp.float32),
        pltpu.VMEM((2, _EMB, _BB), jnp.float32),
        pltpu.SemaphoreType.DMA,
        pltpu.SemaphoreType.DMA,
        pltpu.SemaphoreType.DMA,
        pltpu.SemaphoreType.DMA,
    ],
    compiler_params=pltpu.CompilerParams(
        use_tc_tiling_on_sc=False, needs_layout_passes=False
    ),
)
def _emb_lookup(table_hbm, tok_hbm, out_hbm, idx_v, rows_v, outt_v,
                gsem0, gsem1, osem0, osem1):
    gsem = (gsem0, gsem1)
    osem = (osem0, osem1)
    wid = lax.axis_index("s") * _NC + lax.axis_index("c")
    b0 = wid * _BB

    pltpu.sync_copy(tok_hbm.at[:, pl.ds(b0, _BB)], idx_v)

    iota = lax.iota(jnp.int32, _L)
    rowsel = [iota + (j * _L) for j in range(_BB // _L)]

    def start_gather(t, b):
        pltpu.make_async_copy(
            table_hbm.at[idx_v.at[t]], rows_v.at[b], gsem[b]
        ).start()

    def wait_gather(t, b):
        pltpu.make_async_copy(
            table_hbm.at[idx_v.at[t]], rows_v.at[b], gsem[b]
        ).wait()

    def start_out(t, b):
        pltpu.make_async_copy(
            outt_v.at[b], out_hbm.at[t, :, pl.ds(b0, _BB)], osem[b]
        ).start()

    def wait_out(t, b):
        pltpu.make_async_copy(
            outt_v.at[b], out_hbm.at[t, :, pl.ds(b0, _BB)], osem[b]
        ).wait()

    start_gather(0, 0)
    start_gather(1, 1)

    @pl.loop(0, _T, step=2)
    def _pair(t0):
        for b in range(2):
            t = t0 + b
            wait_gather(t, b)

            @pl.when(t >= 2)
            def _():
                wait_out(t - 2, b)

            rows = rows_v.at[b]
            outt = outt_v.at[b]

            @plsc.parallel_loop(0, _EMB, unroll=4)
            def _col(c):
                colv = jnp.full((_L,), 0, jnp.int32) + c
                for j in range(_BB // _L):
                    v = plsc.load_gather(rows, [rowsel[j], colv])
                    outt[c, pl.ds(j * _L, _L)] = v * _SCALE

            start_out(t, b)

            @pl.when(t + 2 < _T)
            def _():
                start_gather(t + 2, b)

    wait_out(_T - 2, 0)
    wait_out(_T - 1, 1)


def kernel(tokens, table):
    out_t = _emb_lookup(table, tokens.T)
    return jnp.transpose(out_t, (2, 0, 1))
